# Initial kernel scaffold; baseline (speedup 1.0000x reference)
import functools

import jax
import jax.numpy as jnp
from jax import lax
from jax.experimental import pallas as pl
from jax.experimental.pallas import tpu as pltpu
from jax.experimental.pallas import tpu_sc as plsc

B = 16384
EMB = 64
HID = 64
NT = 5

NC = 2
NS = 16
NW = NC * NS
BPW = B // NW
CHUNK = 128
NCHUNK = BPW // CHUNK
TOTAL_CHUNKS = NT * NCHUNK


def _make_sc_gather():
    mesh = plsc.VectorSubcoreMesh(core_axis_name="c", subcore_axis_name="s")

    @functools.partial(
        pl.kernel,
        mesh=mesh,
        out_type=jax.ShapeDtypeStruct((NT, B, EMB), jnp.float32),
        scratch_types=[
            pltpu.VMEM((TOTAL_CHUNKS, CHUNK), jnp.int32),
            pltpu.VMEM((CHUNK, EMB), jnp.float32),
            pltpu.SemaphoreType.DMA,
        ],
    )
    def sc_gather(idx_hbm, pop_hbm, leng_hbm, area_hbm, lon_hbm, lat_hbm,
                  out_hbm, idx_v, rows_v, sem):
        wid = lax.axis_index("s") * NC + lax.axis_index("c")
        base = wid * BPW
        pltpu.sync_copy(idx_hbm.at[wid], idx_v)
        tabs = [pop_hbm, leng_hbm, area_hbm, lon_hbm, lat_hbm]
        for t in range(NT):
            for j in range(NCHUNK):
                c = t * NCHUNK + j
                pltpu.async_copy(tabs[t].at[idx_v.at[c]], rows_v, sem).wait()
                pltpu.sync_copy(rows_v,
                                out_hbm.at[t, pl.ds(base + j * CHUNK, CHUNK)])

    return sc_gather


_sc_gather = _make_sc_gather()


def _mm_body(g_ref, w_ref, b_ref, o_ref):
    acc = jnp.dot(g_ref[0], w_ref[0], preferred_element_type=jnp.float32)
    for t in range(1, NT):
        acc += jnp.dot(g_ref[t], w_ref[t], preferred_element_type=jnp.float32)
    o_ref[...] = acc + b_ref[...]


def _project(g, w_r, b_r):
    BM = 2048
    return pl.pallas_call(
        _mm_body,
        grid=(B // BM,),
        in_specs=[
            pl.BlockSpec((NT, BM, EMB), lambda i: (0, i, 0)),
            pl.BlockSpec((NT, EMB, HID), lambda i: (0, 0, 0)),
            pl.BlockSpec((1, HID), lambda i: (0, 0)),
        ],
        out_specs=pl.BlockSpec((BM, HID), lambda i: (i, 0)),
        out_shape=jax.ShapeDtypeStruct((B, HID), jnp.float32),
    )(g, w_r, b_r)


def kernel(batch_seq_cat, pop_tab, leng_tab, area_tab, lon_tab, lat_tab, W, b):
    idx = batch_seq_cat.astype(jnp.int32).T
    idx = idx.reshape(NT, NW, NCHUNK, CHUNK).transpose(1, 0, 2, 3)
    idx = idx.reshape(NW, TOTAL_CHUNKS, CHUNK)
    g = _sc_gather(idx, pop_tab, leng_tab, area_tab, lon_tab, lat_tab)
    w_r = W.reshape(NT, EMB, HID)
    return _project(g, w_r, b.reshape(1, HID))


# speedup vs baseline: 1.3800x; 1.3800x over previous
import functools

import jax
import jax.numpy as jnp
from jax import lax
from jax.experimental import pallas as pl
from jax.experimental.pallas import tpu as pltpu
from jax.experimental.pallas import tpu_sc as plsc

B = 16384
EMB = 64
HID = 64
NT = 5

NC = 2
NS = 16
NW = NC * NS
BPW = B // NW
CHUNK = 128
NCHUNK = BPW // CHUNK
TOTAL_CHUNKS = NT * NCHUNK


@functools.lru_cache(maxsize=None)
def _make_sc_gather():
    mesh = plsc.VectorSubcoreMesh(core_axis_name="c", subcore_axis_name="s")

    @functools.partial(
        pl.kernel,
        mesh=mesh,
        compiler_params=pltpu.CompilerParams(use_tc_tiling_on_sc=False),
        out_type=jax.ShapeDtypeStruct((NT, B, EMB), jnp.float32),
        scratch_types=[
            pltpu.VMEM((TOTAL_CHUNKS, CHUNK), jnp.int32),
            pltpu.VMEM((CHUNK, EMB), jnp.float32),
            pltpu.SemaphoreType.DMA,
        ],
    )
    def sc_gather(idx_hbm, pop_hbm, leng_hbm, area_hbm, lon_hbm, lat_hbm,
                  out_hbm, idx_v, rows_v, sem):
        wid = lax.axis_index("s") * NC + lax.axis_index("c")
        base = wid * BPW
        pltpu.sync_copy(idx_hbm.at[wid], idx_v)
        tabs = [pop_hbm, leng_hbm, area_hbm, lon_hbm, lat_hbm]
        for t in range(NT):
            for j in range(NCHUNK):
                c = t * NCHUNK + j
                pltpu.async_copy(tabs[t].at[idx_v.at[c]], rows_v, sem).wait()
                pltpu.sync_copy(rows_v,
                                out_hbm.at[t, pl.ds(base + j * CHUNK, CHUNK)])

    return sc_gather


def _mm_body(g_ref, w_ref, b_ref, o_ref):
    acc = jnp.dot(g_ref[0], w_ref[0], preferred_element_type=jnp.float32)
    for t in range(1, NT):
        acc += jnp.dot(g_ref[t], w_ref[t], preferred_element_type=jnp.float32)
    o_ref[...] = acc + b_ref[...]


def _project(g, w_r, b_r):
    BM = 2048
    return pl.pallas_call(
        _mm_body,
        grid=(B // BM,),
        in_specs=[
            pl.BlockSpec((NT, BM, EMB), lambda i: (0, i, 0)),
            pl.BlockSpec((NT, EMB, HID), lambda i: (0, 0, 0)),
            pl.BlockSpec((1, HID), lambda i: (0, 0)),
        ],
        out_specs=pl.BlockSpec((BM, HID), lambda i: (i, 0)),
        out_shape=jax.ShapeDtypeStruct((B, HID), jnp.float32),
    )(g, w_r, b_r)


def kernel(batch_seq_cat, pop_tab, leng_tab, area_tab, lon_tab, lat_tab, W, b):
    idx = batch_seq_cat.astype(jnp.int32).T
    idx = idx.reshape(NT, NW, NCHUNK, CHUNK).transpose(1, 0, 2, 3)
    idx = idx.reshape(NW, TOTAL_CHUNKS, CHUNK)
    g = _make_sc_gather()(idx, pop_tab, leng_tab, area_tab, lon_tab, lat_tab)
    w_r = W.reshape(NT, EMB, HID)
    return _project(g, w_r, b.reshape(1, HID))


# speedup vs baseline: 1.5636x; 1.1330x over previous
import functools

import jax
import jax.numpy as jnp
from jax import lax
from jax.experimental import pallas as pl
from jax.experimental.pallas import tpu as pltpu
from jax.experimental.pallas import tpu_sc as plsc

B = 16384
VOCAB = 100000
EMB = 64
HID = 64
NT = 5

NC = 2
NS = 16
NW = NC * NS
BPW = B // NW
CHUNK = 128
NCHUNK = BPW // CHUNK
TOTAL_CHUNKS = NT * NCHUNK
NBUF = 4


@functools.lru_cache(maxsize=None)
def _make_sc_gather():
    mesh = plsc.VectorSubcoreMesh(core_axis_name="c", subcore_axis_name="s")

    @functools.partial(
        pl.kernel,
        mesh=mesh,
        compiler_params=pltpu.CompilerParams(use_tc_tiling_on_sc=False),
        out_type=jax.ShapeDtypeStruct((NT, B, EMB), jnp.float32),
        scratch_types=[
            pltpu.VMEM((TOTAL_CHUNKS, CHUNK), jnp.int32),
            pltpu.VMEM((NBUF, CHUNK, EMB), jnp.float32),
            pltpu.SemaphoreType.DMA((NBUF,)),
            pltpu.SemaphoreType.DMA((NBUF,)),
        ],
    )
    def sc_gather(idx_hbm, pop_hbm, leng_hbm, area_hbm, lon_hbm, lat_hbm,
                  out_hbm, idx_v, rows_v, gsem, wsem):
        wid = lax.axis_index("s") * NC + lax.axis_index("c")
        base = wid * BPW
        pltpu.sync_copy(idx_hbm.at[wid], idx_v)
        tabs = [pop_hbm, leng_hbm, area_hbm, lon_hbm, lat_hbm]

        def dst(c):
            t, j = divmod(c, NCHUNK)
            return out_hbm.at[t, pl.ds(base + j * CHUNK, CHUNK)]

        gh = [None] * TOTAL_CHUNKS
        wh = [None] * TOTAL_CHUNKS
        for c in range(TOTAL_CHUNKS):
            p = c % NBUF
            if c >= NBUF:
                wh[c - NBUF].wait()
            gh[c] = pltpu.async_copy(
                tabs[c // NCHUNK].at[idx_v.at[c]], rows_v.at[p], gsem.at[p])
            if c >= 1:
                q = (c - 1) % NBUF
                gh[c - 1].wait()
                wh[c - 1] = pltpu.async_copy(rows_v.at[q], dst(c - 1),
                                             wsem.at[q])
        last = TOTAL_CHUNKS - 1
        gh[last].wait()
        wh[last] = pltpu.async_copy(rows_v.at[last % NBUF], dst(last),
                                    wsem.at[last % NBUF])
        for c in range(TOTAL_CHUNKS - NBUF, TOTAL_CHUNKS):
            wh[c].wait()

    return sc_gather


def _mm_body(g_ref, w_ref, b_ref, o_ref):
    acc = b_ref[...].astype(jnp.float32)
    for t in range(NT):
        acc = acc + jnp.dot(g_ref[t], w_ref[t],
                            preferred_element_type=jnp.float32)
    o_ref[...] = acc


def _project(g2, w2, b2):
    BM2 = 1024
    return pl.pallas_call(
        _mm_body,
        grid=(B // 2 // BM2,),
        in_specs=[
            pl.BlockSpec((NT, BM2, 2 * EMB), lambda i: (0, i, 0)),
            pl.BlockSpec((NT, 2 * EMB, 2 * HID), lambda i: (0, 0, 0)),
            pl.BlockSpec((1, 2 * HID), lambda i: (0, 0)),
        ],
        out_specs=pl.BlockSpec((BM2, 2 * HID), lambda i: (i, 0)),
        out_shape=jax.ShapeDtypeStruct((B // 2, 2 * HID), jnp.float32),
    )(g2, w2, b2)


def kernel(batch_seq_cat, pop_tab, leng_tab, area_tab, lon_tab, lat_tab, W, b):
    bsc = batch_seq_cat.astype(jnp.int32)
    idx = bsc.T.reshape(NT, NW, NCHUNK, CHUNK).transpose(1, 0, 2, 3)
    idx = idx.reshape(NW, TOTAL_CHUNKS, CHUNK)
    g = _make_sc_gather()(idx, pop_tab, leng_tab, area_tab, lon_tab, lat_tab)
    g2 = g.reshape(NT, B // 2, 2 * EMB)
    w_r = W.reshape(NT, EMB, HID)
    w2 = jnp.zeros((NT, 2 * EMB, 2 * HID), jnp.float32)
    w2 = w2.at[:, :EMB, :HID].set(w_r).at[:, EMB:, HID:].set(w_r)
    b2 = jnp.concatenate([b, b]).reshape(1, 2 * HID)
    out2 = _project(g2, w2, b2)
    return out2.reshape(B, HID)
